# baseline (speedup 1.0000x reference)
import functools

import jax
import jax.numpy as jnp
from jax import lax
from jax.experimental import pallas as pl
from jax.experimental.pallas import tpu as pltpu
from jax.experimental.pallas import tpu_sc as plsc

SEQ_LEN = 200
BATCH = 1024
DIM = 64
N = SEQ_LEN * BATCH
NUM_WORKERS = 16
B_PER_W = N // NUM_WORKERS
CHUNK = 128
N_CHUNKS = B_PER_W // CHUNK
GATHERS_PER_FILL = 5
ROWS_PER_FILL = CHUNK * GATHERS_PER_FILL
N_FILLS = B_PER_W // ROWS_PER_FILL


def _make_gather():
    mesh = plsc.VectorSubcoreMesh(core_axis_name="c", subcore_axis_name="s",
                                  num_cores=1)

    @functools.partial(
        pl.kernel,
        mesh=mesh,
        out_type=jax.ShapeDtypeStruct((NUM_WORKERS, N_FILLS, ROWS_PER_FILL, DIM),
                                      jnp.float32),
        scratch_types=[
            pltpu.VMEM((N_CHUNKS, CHUNK), jnp.int32),
            pltpu.VMEM((ROWS_PER_FILL, DIM), jnp.float32),
            pltpu.VMEM((ROWS_PER_FILL, DIM), jnp.float32),
            pltpu.SemaphoreType.DMA,
            pltpu.SemaphoreType.DMA,
            pltpu.SemaphoreType.DMA,
            pltpu.SemaphoreType.DMA,
        ],
        compiler_params=pltpu.CompilerParams(use_tc_tiling_on_sc=False),
    )
    def gather(table_hbm, idx_hbm, out_hbm, idx_v, rows0, rows1,
               gsem0, gsem1, wsem0, wsem1):
        wid = lax.axis_index("s") * 1 + lax.axis_index("c")
        pltpu.sync_copy(idx_hbm.at[wid], idx_v)
        rows = (rows0, rows1)
        gsem = (gsem0, gsem1)
        wsem = (wsem0, wsem1)

        def fill_and_drain(g, b):
            hs = [
                pltpu.async_copy(
                    table_hbm.at[idx_v.at[g * GATHERS_PER_FILL + c]],
                    rows[b].at[pl.ds(c * CHUNK, CHUNK)],
                    gsem[b])
                for c in range(GATHERS_PER_FILL)
            ]
            for h in hs:
                h.wait()

        def start_writeout(g, b):
            pltpu.async_copy(rows[b], out_hbm.at[wid, g], wsem[b])

        def wait_writeout(b):
            pltpu.make_async_copy(rows[b], out_hbm.at[wid, 0], wsem[b]).wait()

        fill_and_drain(0, 0)
        start_writeout(0, 0)
        fill_and_drain(1, 1)
        start_writeout(1, 1)

        @pl.loop(2, N_FILLS, step=2)
        def _(g):
            for b in range(2):
                wait_writeout(b)
                fill_and_drain(g + b, b)
                start_writeout(g + b, b)

        wait_writeout(0)
        wait_writeout(1)

    return gather


_gather = _make_gather()


def kernel(source, W):
    idx = source.reshape(NUM_WORKERS, N_CHUNKS, CHUNK)
    out = _gather(W, idx)
    return out.reshape(SEQ_LEN, BATCH, DIM)


# speedup vs baseline: 1.1841x; 1.1841x over previous
import functools

import jax
import jax.numpy as jnp
from jax import lax
from jax.experimental import pallas as pl
from jax.experimental.pallas import tpu as pltpu
from jax.experimental.pallas import tpu_sc as plsc

SEQ_LEN = 200
BATCH = 1024
DIM = 64
DIM_P = 128
N = SEQ_LEN * BATCH
NUM_WORKERS = 32
B_PER_W = N // NUM_WORKERS
CHUNK = 128
N_CHUNKS = B_PER_W // CHUNK
GATHERS_PER_FILL = 2
ROWS_PER_FILL = CHUNK * GATHERS_PER_FILL
N_FILLS = B_PER_W // ROWS_PER_FILL


def _make_gather():
    mesh = plsc.VectorSubcoreMesh(core_axis_name="c", subcore_axis_name="s",
                                  num_cores=2)

    @functools.partial(
        pl.kernel,
        mesh=mesh,
        out_type=jax.ShapeDtypeStruct((NUM_WORKERS, N_FILLS, ROWS_PER_FILL,
                                       DIM_P), jnp.float32),
        scratch_types=[
            pltpu.VMEM((B_PER_W,), jnp.int32),
            pltpu.VMEM((ROWS_PER_FILL, DIM_P), jnp.float32),
            pltpu.VMEM((ROWS_PER_FILL, DIM_P), jnp.float32),
            pltpu.SemaphoreType.DMA,
            pltpu.SemaphoreType.DMA,
            pltpu.SemaphoreType.DMA,
            pltpu.SemaphoreType.DMA,
        ],
        compiler_params=pltpu.CompilerParams(use_tc_tiling_on_sc=True),
    )
    def gather(table_hbm, idx_hbm, out_hbm, idx_v, rows0, rows1,
               gsem0, gsem1, wsem0, wsem1):
        wid = lax.axis_index("s") * 2 + lax.axis_index("c")
        pltpu.sync_copy(idx_hbm.at[pl.ds(wid * B_PER_W, B_PER_W)], idx_v)
        rows = (rows0, rows1)
        gsem = (gsem0, gsem1)
        wsem = (wsem0, wsem1)

        def fill_and_drain(g, b):
            hs = [
                pltpu.async_copy(
                    table_hbm.at[idx_v.at[pl.ds(
                        (g * GATHERS_PER_FILL + c) * CHUNK, CHUNK)]],
                    rows[b].at[pl.ds(c * CHUNK, CHUNK)],
                    gsem[b])
                for c in range(GATHERS_PER_FILL)
            ]
            for h in hs:
                h.wait()

        def start_writeout(g, b):
            pltpu.async_copy(rows[b], out_hbm.at[wid, g], wsem[b])

        def wait_writeout(b):
            pltpu.make_async_copy(rows[b], out_hbm.at[wid, 0], wsem[b]).wait()

        fill_and_drain(0, 0)
        start_writeout(0, 0)
        fill_and_drain(1, 1)
        start_writeout(1, 1)

        @pl.loop(2, N_FILLS - 1, step=2)
        def _(g):
            for b in range(2):
                wait_writeout(b)
                fill_and_drain(g + b, b)
                start_writeout(g + b, b)

        wait_writeout(0)
        fill_and_drain(N_FILLS - 1, 0)
        start_writeout(N_FILLS - 1, 0)

        wait_writeout(0)
        wait_writeout(1)

    return gather


_gather = _make_gather()


def kernel(source, W):
    table = jnp.pad(W, ((0, 0), (0, DIM_P - DIM)))
    idx = source.reshape(N)
    out = _gather(table, idx)
    return out.reshape(SEQ_LEN, BATCH, DIM_P)[:, :, :DIM]
